# baseline (speedup 1.0000x reference)
import jax
import jax.numpy as jnp
from jax.experimental import pallas as pl
from jax.experimental.pallas import tpu as pltpu

LENS_A = [1500, 900, 2100, 1100]
LENS_B = [500, 1100, 300, 900]
D = 256
N_ORIG = sum(LENS_A) + sum(LENS_B)
N = 8448

_ORDERED_LENS = [LENS_A[0], LENS_B[0], LENS_A[1], LENS_B[1],
                 LENS_A[2], LENS_B[2], LENS_A[3], LENS_B[3]]
_OFFSETS = []
_off = 0
for _l in _ORDERED_LENS:
    _OFFSETS.append(_off)
    _off += _l

_SEG_STARTS = [0, 2000, 4000, 6400]

TILE_R = 768
NTILES = N // TILE_R


def _pack_kernel(a0, a1, a2, a3, b0, b1, b2, b3,
                 packed_out, zbuf, in_sems):
    ins = [a0, b0, a1, b1, a2, b2, a3, b3]
    for i, (ref, off, l) in enumerate(zip(ins, _OFFSETS, _ORDERED_LENS)):
        pltpu.make_async_copy(
            ref, packed_out.at[2 * off:2 * (off + l), :], in_sems.at[i]
        ).start()
    zbuf[...] = jnp.zeros((2 * (N - N_ORIG), 128), jnp.float32)
    pltpu.make_async_copy(
        zbuf, packed_out.at[2 * N_ORIG:2 * N, :], in_sems.at[8]
    ).start()
    for i, (ref, off, l) in enumerate(zip(ins, _OFFSETS, _ORDERED_LENS)):
        pltpu.make_async_copy(
            ref, packed_out.at[2 * off:2 * (off + l), :], in_sems.at[i]
        ).wait()
    pltpu.make_async_copy(
        zbuf, packed_out.at[2 * N_ORIG:2 * N, :], in_sems.at[8]
    ).wait()


def _mask_kernel(out_ref):
    t = pl.program_id(0)
    q = jax.lax.broadcasted_iota(jnp.int32, (TILE_R, 1), 0) + t * TILE_R
    k = jax.lax.broadcasted_iota(jnp.int32, (1, N), 1)

    def seg_id(p):
        s = jnp.zeros(p.shape, jnp.int32)
        for b in _SEG_STARTS[1:]:
            s = s + (p >= b).astype(jnp.int32)
        return s

    out_ref[...] = (seg_id(q) == seg_id(k)) & (q < N_ORIG) & (k < N_ORIG)


def kernel(a0, a1, a2, a3, b0, b1, b2, b3):
    views = [jnp.reshape(x, (2 * x.shape[1], 128))
             for x in (a0, a1, a2, a3, b0, b1, b2, b3)]
    packed2d = pl.pallas_call(
        _pack_kernel,
        in_specs=[pl.BlockSpec(memory_space=pl.ANY)] * 8,
        out_specs=pl.BlockSpec(memory_space=pl.ANY),
        out_shape=jax.ShapeDtypeStruct((2 * N, 128), jnp.float32),
        scratch_shapes=[
            pltpu.VMEM((2 * (N - N_ORIG), 128), jnp.float32),
            pltpu.SemaphoreType.DMA((9,)),
        ],
    )(*views)

    mask = pl.pallas_call(
        _mask_kernel,
        grid=(NTILES,),
        out_specs=pl.BlockSpec((TILE_R, N), lambda t: (t, 0)),
        out_shape=jax.ShapeDtypeStruct((N, N), jnp.bool_),
    )()
    return jnp.reshape(packed2d, (1, N, D)), mask


# speedup vs baseline: 2.2021x; 2.2021x over previous
import jax
import jax.numpy as jnp
from jax.experimental import pallas as pl

LENS_A = [1500, 900, 2100, 1100]
LENS_B = [500, 1100, 300, 900]
D = 256
N_ORIG = sum(LENS_A) + sum(LENS_B)
N = 8448

_ORDERED_LENS = [LENS_A[0], LENS_B[0], LENS_A[1], LENS_B[1],
                 LENS_A[2], LENS_B[2], LENS_A[3], LENS_B[3]]
_OFFSETS = []
_off = 0
for _l in _ORDERED_LENS:
    _OFFSETS.append(_off)
    _off += _l

_SEG_STARTS = [0, 2000, 4000, 6400]

TILE_R = 384
NTILES = N // TILE_R
PACK_TILE = 2 * N // NTILES

_IN_ORDER = [0, 4, 1, 5, 2, 6, 3, 7]


def _fused_kernel(a0, a1, a2, a3, b0, b1, b2, b3, mask_ref, packed_ref):
    t = pl.program_id(0)

    q = jax.lax.broadcasted_iota(jnp.int32, (TILE_R, 1), 0) + t * TILE_R
    k = jax.lax.broadcasted_iota(jnp.int32, (1, N), 1)

    def seg_id(p):
        s = jnp.zeros(p.shape, jnp.int32)
        for b in _SEG_STARTS[1:]:
            s = s + (p >= b).astype(jnp.int32)
        return s

    mask_ref[...] = (seg_id(q) == seg_id(k)) & (q < N_ORIG) & (k < N_ORIG)

    ins = [a0, a1, a2, a3, b0, b1, b2, b3]
    for step in range(NTILES):
        lo, hi = PACK_TILE * step, PACK_TILE * (step + 1)

        def _copies(lo=lo, hi=hi):
            for idx, off, l in zip(_IN_ORDER, _OFFSETS, _ORDERED_LENS):
                s0, s1 = max(lo, 2 * off), min(hi, 2 * (off + l))
                if s0 < s1:
                    packed_ref[s0 - lo:s1 - lo, :] = (
                        ins[idx][s0 - 2 * off:s1 - 2 * off, :])
            z0, z1 = max(lo, 2 * N_ORIG), hi
            if z0 < z1:
                packed_ref[z0 - lo:z1 - lo, :] = jnp.zeros(
                    (z1 - z0, 128), jnp.float32)

        pl.when(t == step)(_copies)


def kernel(a0, a1, a2, a3, b0, b1, b2, b3):
    views = [jnp.reshape(x, (2 * x.shape[1], 128))
             for x in (a0, a1, a2, a3, b0, b1, b2, b3)]
    full_specs = [
        pl.BlockSpec((2 * x.shape[1], 128), lambda t: (0, 0))
        for x in (a0, a1, a2, a3, b0, b1, b2, b3)
    ]
    mask, packed2d = pl.pallas_call(
        _fused_kernel,
        grid=(NTILES,),
        in_specs=full_specs,
        out_specs=(
            pl.BlockSpec((TILE_R, N), lambda t: (t, 0)),
            pl.BlockSpec((PACK_TILE, 128), lambda t: (t, 0)),
        ),
        out_shape=(
            jax.ShapeDtypeStruct((N, N), jnp.bool_),
            jax.ShapeDtypeStruct((2 * N, 128), jnp.float32),
        ),
    )(*views)
    return jnp.reshape(packed2d, (1, N, D)), mask
